# baseline (speedup 1.0000x reference)
import functools

import jax
import jax.numpy as jnp
from jax import lax
from jax.experimental import pallas as pl
from jax.experimental.pallas import tpu as pltpu
from jax.experimental.pallas import tpu_sc as plsc

_BATCH = 16384
_NC, _NS = 2, 16
_NW = _NC * _NS
_B_PER_W = _BATCH // _NW


def _make_gather():
    mesh = plsc.VectorSubcoreMesh(core_axis_name="c", subcore_axis_name="s")

    @functools.partial(
        pl.kernel,
        mesh=mesh,
        out_type=jax.ShapeDtypeStruct((_BATCH,), jnp.float32),
        scratch_types=[
            pltpu.VMEM((_B_PER_W,), jnp.int32),
            pltpu.VMEM((_B_PER_W,), jnp.float32),
            pltpu.SemaphoreType.DMA,
            pltpu.SemaphoreType.DMA,
            pltpu.SemaphoreType.DMA,
            pltpu.SemaphoreType.DMA,
        ],
    )
    def gather_kernel(
        v_hbm, idx_hbm, out_hbm, idx_v, out_v, g0, g1, s0, s1
    ):
        wid = lax.axis_index("c") * _NS + lax.axis_index("s")
        base = wid * _B_PER_W
        half = _B_PER_W // 2
        pltpu.sync_copy(idx_hbm.at[pl.ds(base, half)], idx_v.at[pl.ds(0, half)])
        c0 = pltpu.async_copy(
            v_hbm.at[idx_v.at[pl.ds(0, half)]], out_v.at[pl.ds(0, half)], g0
        )
        pltpu.sync_copy(
            idx_hbm.at[pl.ds(base + half, half)], idx_v.at[pl.ds(half, half)]
        )
        c1 = pltpu.async_copy(
            v_hbm.at[idx_v.at[pl.ds(half, half)]],
            out_v.at[pl.ds(half, half)],
            g1,
        )
        c0.wait()
        w0 = pltpu.async_copy(
            out_v.at[pl.ds(0, half)], out_hbm.at[pl.ds(base, half)], s0
        )
        c1.wait()
        w1 = pltpu.async_copy(
            out_v.at[pl.ds(half, half)],
            out_hbm.at[pl.ds(base + half, half)],
            s1,
        )
        w0.wait()
        w1.wait()

    return gather_kernel


_GATHER = _make_gather()


def kernel(v, indices):
    return _GATHER(v, indices)


# speedup vs baseline: 1.0144x; 1.0144x over previous
import functools

import jax
import jax.numpy as jnp
from jax import lax
from jax.experimental import pallas as pl
from jax.experimental.pallas import tpu as pltpu
from jax.experimental.pallas import tpu_sc as plsc

_BATCH = 16384
_NC, _NS = 2, 16
_NW = _NC * _NS
_B_PER_W = _BATCH // _NW


def _make_gather():
    mesh = plsc.VectorSubcoreMesh(core_axis_name="c", subcore_axis_name="s")

    @functools.partial(
        pl.kernel,
        mesh=mesh,
        out_type=jax.ShapeDtypeStruct((_BATCH,), jnp.float32),
        scratch_types=[
            pltpu.VMEM((_B_PER_W,), jnp.int32),
            pltpu.VMEM((_B_PER_W,), jnp.float32),
            pltpu.SemaphoreType.DMA,
            pltpu.SemaphoreType.DMA,
            pltpu.SemaphoreType.DMA,
            pltpu.SemaphoreType.DMA,
        ],
    )
    def gather_kernel(
        v_hbm, idx_hbm, out_hbm, idx_v, out_v, g0, g1, s0, s1
    ):
        wid = lax.axis_index("c") * _NS + lax.axis_index("s")
        base = wid * _B_PER_W
        half = _B_PER_W // 2
        pltpu.sync_copy(idx_hbm.at[pl.ds(base, _B_PER_W)], idx_v)
        c0 = pltpu.async_copy(
            v_hbm.at[idx_v.at[pl.ds(0, half)]], out_v.at[pl.ds(0, half)], g0
        )
        c1 = pltpu.async_copy(
            v_hbm.at[idx_v.at[pl.ds(half, half)]],
            out_v.at[pl.ds(half, half)],
            g1,
        )
        c0.wait()
        w0 = pltpu.async_copy(
            out_v.at[pl.ds(0, half)], out_hbm.at[pl.ds(base, half)], s0
        )
        c1.wait()
        w1 = pltpu.async_copy(
            out_v.at[pl.ds(half, half)],
            out_hbm.at[pl.ds(base + half, half)],
            s1,
        )
        w0.wait()
        w1.wait()

    return gather_kernel


_GATHER = _make_gather()


def kernel(v, indices):
    return _GATHER(v, indices)
